# baseline (device time: 313350 ns/iter reference)
import jax
import jax.numpy as jnp
from jax import lax
from jax.experimental import pallas as pl
from jax.experimental.pallas import tpu as pltpu

N_DEV = 4
SQ = 2048
DM = 1024
HEADS = 8
DH = 128
QC = 256
N_QC = SQ // QC
BLK = 64
SCALE = 0.08838834764831843


def _body(x_ref, wq_ref, wo_ref, k_hbm, v_hbm, out_ref,
          comm_ref, q_scr, ctx_scr, kbuf, vbuf,
          send_sems, recv_sems, k_sems, v_sems):
    my = lax.axis_index("i")
    left = lax.rem(my + N_DEV - 1, N_DEV)
    right = lax.rem(my + 1, N_DEV)

    barrier_sem = pltpu.get_barrier_semaphore()
    for nbr in (left, right):
        pl.semaphore_signal(barrier_sem, inc=1, device_id=(nbr,),
                            device_id_type=pl.DeviceIdType.MESH)
    pl.semaphore_wait(barrier_sem, 2)

    comm_ref[0, 0] = wq_ref[...]
    comm_ref[0, 1] = wo_ref[...]

    for h in range(N_DEV):
        slot = h % 2
        g = lax.rem(my - h + N_DEV, N_DEV)
        g8 = g * HEADS

        rdma = None
        if h < N_DEV - 1:
            rdma = pltpu.make_async_remote_copy(
                src_ref=comm_ref.at[slot],
                dst_ref=comm_ref.at[1 - slot],
                send_sem=send_sems.at[slot],
                recv_sem=recv_sems.at[1 - slot],
                device_id=(right,),
                device_id_type=pl.DeviceIdType.MESH,
            )
            rdma.start()

        q = jnp.dot(x_ref[...], comm_ref[slot, 0],
                    preferred_element_type=jnp.float32)
        q_scr[...] = q.astype(jnp.bfloat16)

        pltpu.make_async_copy(k_hbm.at[g8], kbuf.at[0], k_sems.at[0]).start()
        pltpu.make_async_copy(v_hbm.at[g8], vbuf.at[0], v_sems.at[0]).start()

        def head_body(hh, _, g8=g8):
            cur = lax.rem(hh, 2)
            nxt = lax.rem(hh + 1, 2)
            pltpu.make_async_copy(
                k_hbm.at[g8 + hh], kbuf.at[cur], k_sems.at[cur]).wait()
            pltpu.make_async_copy(
                v_hbm.at[g8 + hh], vbuf.at[cur], v_sems.at[cur]).wait()

            @pl.when(hh < HEADS - 1)
            def _():
                pltpu.make_async_copy(
                    k_hbm.at[g8 + hh + 1], kbuf.at[nxt], k_sems.at[nxt]).start()
                pltpu.make_async_copy(
                    v_hbm.at[g8 + hh + 1], vbuf.at[nxt], v_sems.at[nxt]).start()

            kk = kbuf[cur]
            vv = vbuf[cur]
            col0 = hh * DH
            for qc in range(N_QC):
                kv = (qc + 1) * QC
                qh = q_scr[pl.ds(qc * QC, QC), pl.ds(col0, DH)]
                s = lax.dot_general(
                    qh, kk[:kv, :], (((1,), (1,)), ((), ())),
                    preferred_element_type=jnp.float32) * SCALE
                rows = qc * QC + lax.broadcasted_iota(jnp.int32, (QC, kv), 0)
                cols = lax.broadcasted_iota(jnp.int32, (QC, kv), 1)
                s = jnp.where((cols // BLK) <= (rows // BLK), s,
                              jnp.float32(-1e9))
                m = jnp.max(s, axis=-1, keepdims=True)
                e = jnp.exp(s - m)
                den = jnp.sum(e, axis=-1, keepdims=True)
                p = (e / den).astype(jnp.bfloat16)
                ctx = jnp.dot(p, vv[:kv, :],
                              preferred_element_type=jnp.float32)
                ctx_scr[pl.ds(qc * QC, QC), pl.ds(col0, DH)] = (
                    ctx.astype(jnp.bfloat16))
            return 0

        lax.fori_loop(0, HEADS, head_body, 0)

        partial = jnp.dot(ctx_scr[...], comm_ref[slot, 1],
                          preferred_element_type=jnp.float32)
        if h == 0:
            out_ref[...] = partial
        else:
            out_ref[...] = out_ref[...] + partial

        if h < N_DEV - 1:
            rdma.wait()


def kernel(x, Wq, K_ext, V_ext, Wo):
    my = lax.axis_index("i")
    xb = x[0].astype(jnp.bfloat16)
    wq = Wq.astype(jnp.bfloat16)
    wo = Wo.astype(jnp.bfloat16)
    k_loc = lax.dynamic_index_in_dim(K_ext, my, 0, keepdims=False)
    v_loc = lax.dynamic_index_in_dim(V_ext, my, 0, keepdims=False)
    kh = jnp.transpose(k_loc, (1, 0, 2)).astype(jnp.bfloat16)
    vh = jnp.transpose(v_loc, (1, 0, 2)).astype(jnp.bfloat16)

    out = pl.pallas_call(
        _body,
        out_shape=jax.ShapeDtypeStruct((SQ, DM), jnp.float32),
        in_specs=[
            pl.BlockSpec(memory_space=pltpu.VMEM),
            pl.BlockSpec(memory_space=pltpu.VMEM),
            pl.BlockSpec(memory_space=pltpu.VMEM),
            pl.BlockSpec(memory_space=pl.ANY),
            pl.BlockSpec(memory_space=pl.ANY),
        ],
        out_specs=pl.BlockSpec(memory_space=pltpu.VMEM),
        scratch_shapes=[
            pltpu.VMEM((2, 2, DM, DM), jnp.bfloat16),
            pltpu.VMEM((SQ, DM), jnp.bfloat16),
            pltpu.VMEM((SQ, DM), jnp.bfloat16),
            pltpu.VMEM((2, SQ, DH), jnp.bfloat16),
            pltpu.VMEM((2, SQ, DH), jnp.bfloat16),
            pltpu.SemaphoreType.DMA((2,)),
            pltpu.SemaphoreType.DMA((2,)),
            pltpu.SemaphoreType.DMA((2,)),
            pltpu.SemaphoreType.DMA((2,)),
        ],
        compiler_params=pltpu.CompilerParams(collective_id=0),
    )(xb, wq, wo, kh, vh)
    return out.reshape(1, SQ, DM)


# device time: 208556 ns/iter; 1.5025x vs baseline; 1.5025x over previous
import jax
import jax.numpy as jnp
from jax import lax
from jax.experimental import pallas as pl
from jax.experimental.pallas import tpu as pltpu

N_DEV = 4
SQ = 2048
DM = 1024
HEADS = 8
DH = 128
QC = 256
N_QC = SQ // QC
BLK = 64
SCALE = 0.08838834764831843
NEG = -1e9


def _body(x_ref, wq_ref, wo_ref, k_hbm, v_hbm, out_ref,
          comm_ref, q_scr, ctx_scr, kbuf, vbuf,
          send_sems, recv_sems, k_sems, v_sems):
    my = lax.axis_index("i")
    left = lax.rem(my + N_DEV - 1, N_DEV)
    right = lax.rem(my + 1, N_DEV)

    def kv_dma(hbm, buf, sems, hidx, bslot):
        return pltpu.make_async_copy(
            hbm.at[my, :, hidx, :], buf.at[bslot], sems.at[bslot])

    barrier_sem = pltpu.get_barrier_semaphore()
    for nbr in (left, right):
        pl.semaphore_signal(barrier_sem, inc=1, device_id=(nbr,),
                            device_id_type=pl.DeviceIdType.MESH)
    pl.semaphore_wait(barrier_sem, 2)

    comm_ref[0, 0] = wq_ref[...]
    comm_ref[0, 1] = wo_ref[...]

    drows = lax.broadcasted_iota(jnp.int32, (QC, QC), 0) // BLK
    dcols = lax.broadcasted_iota(jnp.int32, (QC, QC), 1) // BLK
    dmask = dcols <= drows

    for h in range(N_DEV):
        slot = h % 2
        g = lax.rem(my - h + N_DEV, N_DEV)
        g8 = g * HEADS

        rdma = None
        if h < N_DEV - 1:
            rdma = pltpu.make_async_remote_copy(
                src_ref=comm_ref.at[slot],
                dst_ref=comm_ref.at[1 - slot],
                send_sem=send_sems.at[slot],
                recv_sem=recv_sems.at[1 - slot],
                device_id=(right,),
                device_id_type=pl.DeviceIdType.MESH,
            )
            rdma.start()

        kv_dma(k_hbm, kbuf, k_sems, g8, 0).start()
        kv_dma(v_hbm, vbuf, v_sems, g8, 0).start()

        q = jnp.dot(x_ref[...], comm_ref[slot, 0],
                    preferred_element_type=jnp.float32)
        q_scr[...] = (q * SCALE).astype(jnp.bfloat16)

        def head_body(hh, _, g8=g8):
            cur = lax.rem(hh, 2)
            nxt = lax.rem(hh + 1, 2)
            kv_dma(k_hbm, kbuf, k_sems, g8 + hh, cur).wait()
            kv_dma(v_hbm, vbuf, v_sems, g8 + hh, cur).wait()

            @pl.when(hh < HEADS - 1)
            def _():
                kv_dma(k_hbm, kbuf, k_sems, g8 + hh + 1, nxt).start()
                kv_dma(v_hbm, vbuf, v_sems, g8 + hh + 1, nxt).start()

            kk = kbuf[cur].astype(jnp.bfloat16)
            vv = vbuf[cur].astype(jnp.bfloat16)
            col0 = hh * DH
            for qc in range(N_QC):
                kv0 = qc * QC
                qh = q_scr[pl.ds(kv0, QC), pl.ds(col0, DH)]
                sd = lax.dot_general(
                    qh, kk[kv0:kv0 + QC, :], (((1,), (1,)), ((), ())),
                    preferred_element_type=jnp.float32)
                ed = jnp.exp(jnp.where(dmask, sd, NEG))
                if qc == 0:
                    den = jnp.sum(ed, axis=-1, keepdims=True)
                    pd = (ed * (1.0 / den)).astype(jnp.bfloat16)
                    ctx = jnp.dot(pd, vv[:QC, :],
                                  preferred_element_type=jnp.float32)
                else:
                    sl = lax.dot_general(
                        qh, kk[:kv0, :], (((1,), (1,)), ((), ())),
                        preferred_element_type=jnp.float32)
                    el = jnp.exp(sl)
                    den = (jnp.sum(el, axis=-1, keepdims=True)
                           + jnp.sum(ed, axis=-1, keepdims=True))
                    r = 1.0 / den
                    pl_ = (el * r).astype(jnp.bfloat16)
                    pd = (ed * r).astype(jnp.bfloat16)
                    ctx = (jnp.dot(pl_, vv[:kv0, :],
                                   preferred_element_type=jnp.float32)
                           + jnp.dot(pd, vv[kv0:kv0 + QC, :],
                                     preferred_element_type=jnp.float32))
                ctx_scr[pl.ds(kv0, QC), pl.ds(col0, DH)] = (
                    ctx.astype(jnp.bfloat16))
            return 0

        lax.fori_loop(0, HEADS, head_body, 0)

        partial = jnp.dot(ctx_scr[...], comm_ref[slot, 1],
                          preferred_element_type=jnp.float32)
        if h == 0:
            out_ref[...] = partial
        else:
            out_ref[...] = out_ref[...] + partial

        if h < N_DEV - 1:
            rdma.wait()


def kernel(x, Wq, K_ext, V_ext, Wo):
    xb = x[0].astype(jnp.bfloat16)
    wq = Wq.astype(jnp.bfloat16)
    wo = Wo.astype(jnp.bfloat16)

    out = pl.pallas_call(
        _body,
        out_shape=jax.ShapeDtypeStruct((SQ, DM), jnp.float32),
        in_specs=[
            pl.BlockSpec(memory_space=pltpu.VMEM),
            pl.BlockSpec(memory_space=pltpu.VMEM),
            pl.BlockSpec(memory_space=pltpu.VMEM),
            pl.BlockSpec(memory_space=pl.ANY),
            pl.BlockSpec(memory_space=pl.ANY),
        ],
        out_specs=pl.BlockSpec(memory_space=pltpu.VMEM),
        scratch_shapes=[
            pltpu.VMEM((2, 2, DM, DM), jnp.bfloat16),
            pltpu.VMEM((SQ, DM), jnp.bfloat16),
            pltpu.VMEM((SQ, DM), jnp.bfloat16),
            pltpu.VMEM((2, SQ, DH), jnp.float32),
            pltpu.VMEM((2, SQ, DH), jnp.float32),
            pltpu.SemaphoreType.DMA((2,)),
            pltpu.SemaphoreType.DMA((2,)),
            pltpu.SemaphoreType.DMA((2,)),
            pltpu.SemaphoreType.DMA((2,)),
        ],
        compiler_params=pltpu.CompilerParams(collective_id=0),
    )(xb, wq, wo, K_ext, V_ext)
    return out.reshape(1, SQ, DM)


# device time: 178955 ns/iter; 1.7510x vs baseline; 1.1654x over previous
import jax
import jax.numpy as jnp
from jax import lax
from jax.experimental import pallas as pl
from jax.experimental.pallas import tpu as pltpu

N_DEV = 4
SQ = 2048
DM = 1024
HEADS = 8
DH = 128
QC = 256
N_QC = SQ // QC
BLK = 64
SCALE = 0.08838834764831843
NEG = -1e9


def _body(x_ref, wq_ref, wo_ref, k_hbm, v_hbm, out_ref,
          comm_ref, q_scr, ctx_scr, kbuf, vbuf,
          send_sems, recv_sems, k_sems, v_sems):
    my = lax.axis_index("i")
    left = lax.rem(my + N_DEV - 1, N_DEV)
    right = lax.rem(my + 1, N_DEV)

    def kv_dma(hbm, buf, sems, hidx, bslot):
        return pltpu.make_async_copy(
            hbm.at[my, :, hidx, :], buf.at[bslot], sems.at[bslot])

    def weights_rdma(src_slot, dst_slot, sem_i, dst_dev):
        return pltpu.make_async_remote_copy(
            src_ref=comm_ref.at[src_slot],
            dst_ref=comm_ref.at[dst_slot],
            send_sem=send_sems.at[sem_i],
            recv_sem=recv_sems.at[dst_slot - 1],
            device_id=(dst_dev,),
            device_id_type=pl.DeviceIdType.MESH,
        )

    barrier_sem = pltpu.get_barrier_semaphore()
    for nbr in (left, right):
        pl.semaphore_signal(barrier_sem, inc=1, device_id=(nbr,),
                            device_id_type=pl.DeviceIdType.MESH)
    pl.semaphore_wait(barrier_sem, 2)

    comm_ref[0, 0] = wq_ref[...]
    comm_ref[0, 1] = wo_ref[...]

    rd_r = weights_rdma(0, 1, 0, right)
    rd_l = weights_rdma(0, 2, 1, left)
    rd_r.start()
    rd_l.start()

    drows = lax.broadcasted_iota(jnp.int32, (QC, QC), 0) // BLK
    dcols = lax.broadcasted_iota(jnp.int32, (QC, QC), 1) // BLK
    dmask = dcols <= drows

    groups = [my, left, right, lax.rem(my + 2, N_DEV)]

    for h in range(N_DEV):
        slot = h
        g = groups[h]
        g8 = g * HEADS

        kv_dma(k_hbm, kbuf, k_sems, g8, 0).start()
        kv_dma(v_hbm, vbuf, v_sems, g8, 0).start()

        q = jnp.dot(x_ref[...], comm_ref[slot, 0],
                    preferred_element_type=jnp.float32)
        q_scr[...] = q.astype(jnp.bfloat16)

        def head_body(hh, _, g8=g8):
            cur = lax.rem(hh, 2)
            nxt = lax.rem(hh + 1, 2)
            kv_dma(k_hbm, kbuf, k_sems, g8 + hh, cur).wait()
            kv_dma(v_hbm, vbuf, v_sems, g8 + hh, cur).wait()

            @pl.when(hh < HEADS - 1)
            def _():
                kv_dma(k_hbm, kbuf, k_sems, g8 + hh + 1, nxt).start()
                kv_dma(v_hbm, vbuf, v_sems, g8 + hh + 1, nxt).start()

            kk = kbuf[cur].astype(jnp.bfloat16)
            vv = vbuf[cur].astype(jnp.bfloat16)
            col0 = hh * DH
            for qc in range(N_QC):
                kv0 = qc * QC
                qh = q_scr[pl.ds(kv0, QC), pl.ds(col0, DH)]
                sd = lax.dot_general(
                    qh, kk[kv0:kv0 + QC, :], (((1,), (1,)), ((), ())),
                    preferred_element_type=jnp.float32)
                ed = jnp.exp(jnp.where(dmask, sd, NEG))
                den = jnp.sum(ed, axis=-1, keepdims=True)
                ctx = jnp.dot(ed.astype(jnp.bfloat16), vv[kv0:kv0 + QC, :],
                              preferred_element_type=jnp.float32)
                if qc > 0:
                    sl = lax.dot_general(
                        qh, kk[:kv0, :], (((1,), (1,)), ((), ())),
                        preferred_element_type=jnp.float32)
                    el = jnp.exp(sl)
                    den = den + jnp.sum(el, axis=-1, keepdims=True)
                    ctx = ctx + jnp.dot(el.astype(jnp.bfloat16), vv[:kv0, :],
                                        preferred_element_type=jnp.float32)
                ctx_scr[pl.ds(kv0, QC), pl.ds(col0, DH)] = (
                    (ctx * (1.0 / den)).astype(jnp.bfloat16))
            return 0

        lax.fori_loop(0, HEADS, head_body, 0)

        partial = jnp.dot(ctx_scr[...], comm_ref[slot, 1],
                          preferred_element_type=jnp.float32)
        if h == 0:
            out_ref[...] = partial
        else:
            out_ref[...] = out_ref[...] + partial

        if h == 0:
            rd_r.wait()
            rd_f = weights_rdma(1, 3, 2, right)
            rd_f.start()
        elif h == 1:
            rd_l.wait()
        elif h == 2:
            rd_f.wait()


def kernel(x, Wq, K_ext, V_ext, Wo):
    xb = (x[0] * SCALE).astype(jnp.bfloat16)
    wq = Wq.astype(jnp.bfloat16)
    wo = Wo.astype(jnp.bfloat16)

    out = pl.pallas_call(
        _body,
        out_shape=jax.ShapeDtypeStruct((SQ, DM), jnp.float32),
        in_specs=[
            pl.BlockSpec(memory_space=pltpu.VMEM),
            pl.BlockSpec(memory_space=pltpu.VMEM),
            pl.BlockSpec(memory_space=pltpu.VMEM),
            pl.BlockSpec(memory_space=pl.ANY),
            pl.BlockSpec(memory_space=pl.ANY),
        ],
        out_specs=pl.BlockSpec(memory_space=pltpu.VMEM),
        scratch_shapes=[
            pltpu.VMEM((4, 2, DM, DM), jnp.bfloat16),
            pltpu.VMEM((SQ, DM), jnp.bfloat16),
            pltpu.VMEM((SQ, DM), jnp.bfloat16),
            pltpu.VMEM((2, SQ, DH), jnp.float32),
            pltpu.VMEM((2, SQ, DH), jnp.float32),
            pltpu.SemaphoreType.DMA((3,)),
            pltpu.SemaphoreType.DMA((3,)),
            pltpu.SemaphoreType.DMA((2,)),
            pltpu.SemaphoreType.DMA((2,)),
        ],
        compiler_params=pltpu.CompilerParams(
            collective_id=0, vmem_limit_bytes=56 * 1024 * 1024),
    )(xb, wq, wo, K_ext, V_ext)
    return out.reshape(1, SQ, DM)
